# Initial kernel scaffold; baseline (speedup 1.0000x reference)
import jax
import jax.numpy as jnp
from jax.experimental import pallas as pl


def kernel(x, edge_index, edge_rbf, lin1_w, lin1_b, lin2_w, lin2_b, fn1_w, fn1_b, fn2_w, fn2_b):
    raise NotImplementedError("write your pallas kernel here")


# speedup vs baseline: 2.4780x; 2.4780x over previous
import functools

import jax
import jax.numpy as jnp
from jax import lax
from jax.experimental import pallas as pl
from jax.experimental.pallas import tpu as pltpu
from jax.experimental.pallas import tpu_sc as plsc

N_NODES = 10000
N_EDGES = 320000
D = 128
RBF = 16

NC = 2
NS = 16
NW = NC * NS
EPW = N_EDGES // NW
C = 80
K = EPW // C
N_PAD = 10240
ROWS_PER_TILE = N_PAD // NS

EB = 2560


def _linear_kernel(x_ref, w_ref, b_ref, o_ref):
    o_ref[...] = (
        jnp.dot(x_ref[...], w_ref[...], preferred_element_type=jnp.float32)
        + b_ref[...]
    )


def _node_linear(x, w, b):
    return pl.pallas_call(
        _linear_kernel,
        out_shape=jax.ShapeDtypeStruct((x.shape[0], w.shape[1]), jnp.float32),
    )(x, w, b.reshape(1, -1))


def _sum_linear_kernel(p_ref, w_ref, b_ref, o_ref):
    s = p_ref[0, :N_NODES, :] + p_ref[1, :N_NODES, :]
    o_ref[...] = (
        jnp.dot(s, w_ref[...], preferred_element_type=jnp.float32) + b_ref[...]
    )


def _final_linear(partials, w, b):
    return pl.pallas_call(
        _sum_linear_kernel,
        out_shape=jax.ShapeDtypeStruct((N_NODES, w.shape[1]), jnp.float32),
    )(partials, w, b.reshape(1, -1))


def _filter_kernel(rbf_ref, w1_ref, b1_ref, w2_ref, b2_ref, o_ref):
    h = (
        jnp.dot(rbf_ref[...], w1_ref[...], preferred_element_type=jnp.float32)
        + b1_ref[...]
    )
    h = h * jax.nn.sigmoid(h)
    o_ref[...] = (
        jnp.dot(h, w2_ref[...], preferred_element_type=jnp.float32) + b2_ref[...]
    )


def _filter_net(edge_rbf, fn1_w, fn1_b, fn2_w, fn2_b):
    grid = (N_EDGES // EB,)
    return pl.pallas_call(
        _filter_kernel,
        grid=grid,
        in_specs=[
            pl.BlockSpec((EB, RBF), lambda i: (i, 0)),
            pl.BlockSpec((RBF, D), lambda i: (0, 0)),
            pl.BlockSpec((1, D), lambda i: (0, 0)),
            pl.BlockSpec((D, D), lambda i: (0, 0)),
            pl.BlockSpec((1, D), lambda i: (0, 0)),
        ],
        out_specs=pl.BlockSpec((EB, D), lambda i: (i, 0)),
        out_shape=jax.ShapeDtypeStruct((N_EDGES, D), jnp.float32),
    )(edge_rbf, fn1_w, fn1_b.reshape(1, D), fn2_w, fn2_b.reshape(1, D))


def _sc_body(x1_hbm, col_hbm, row_hbm, filt_hbm, zeros_hbm, out_hbm,
             colv, rowv, gbuf, fbuf, acc, sem):
    cid = lax.axis_index("c")
    sid = lax.axis_index("s")
    wid = sid * NC + cid

    pltpu.sync_copy(
        zeros_hbm.at[pl.ds(sid * ROWS_PER_TILE, ROWS_PER_TILE)],
        acc.at[pl.ds(sid * ROWS_PER_TILE, ROWS_PER_TILE)],
    )
    plsc.subcore_barrier()

    base0 = wid * EPW

    def chunk_body(i, carry):
        base = base0 + i * C
        pltpu.sync_copy(col_hbm.at[pl.ds(base, C)], colv)
        pltpu.sync_copy(row_hbm.at[pl.ds(base, C)], rowv)
        pltpu.async_copy(x1_hbm.at[colv], gbuf, sem).wait()
        pltpu.sync_copy(filt_hbm.at[pl.ds(base, C)], fbuf)

        def mul_row(r, c2):
            for j in range(D // 16):
                s = pl.ds(j * 16, 16)
                gbuf[r, s] = gbuf[r, s] * fbuf[r, s]
            return c2

        lax.fori_loop(0, C, mul_row, 0)
        pltpu.sync_copy(gbuf, acc.at[rowv], add=True)
        return carry

    lax.fori_loop(0, K, chunk_body, 0)
    plsc.subcore_barrier()

    pltpu.sync_copy(
        acc.at[pl.ds(sid * ROWS_PER_TILE, ROWS_PER_TILE)],
        out_hbm.at[cid, pl.ds(sid * ROWS_PER_TILE, ROWS_PER_TILE)],
    )


def _sc_gather_mul_scatter(x1, col, row, filt, zeros):
    mesh = plsc.VectorSubcoreMesh(core_axis_name="c", subcore_axis_name="s")
    f = functools.partial(
        pl.kernel,
        mesh=mesh,
        out_type=jax.ShapeDtypeStruct((NC, N_PAD, D), jnp.float32),
        scratch_types=[
            pltpu.VMEM((C,), jnp.int32),
            pltpu.VMEM((C,), jnp.int32),
            pltpu.VMEM((C, D), jnp.float32),
            pltpu.VMEM((C, D), jnp.float32),
            pltpu.VMEM_SHARED((N_PAD, D), jnp.float32),
            pltpu.SemaphoreType.DMA,
        ],
    )(_sc_body)
    return f(x1, col, row, filt, zeros)


def kernel(x, edge_index, edge_rbf, lin1_w, lin1_b, lin2_w, lin2_b,
           fn1_w, fn1_b, fn2_w, fn2_b):
    ei = edge_index.astype(jnp.int32)
    row = ei[0]
    col = ei[1]
    x1 = _node_linear(x, lin1_w, lin1_b)
    filt = _filter_net(edge_rbf, fn1_w, fn1_b, fn2_w, fn2_b)
    zeros = jnp.zeros((N_PAD, D), dtype=jnp.float32)
    partials = _sc_gather_mul_scatter(x1, col, row, filt, zeros)
    return _final_linear(partials, lin2_w, lin2_b)


# speedup vs baseline: 4.3263x; 1.7459x over previous
import functools

import jax
import jax.numpy as jnp
from jax import lax
from jax.experimental import pallas as pl
from jax.experimental.pallas import tpu as pltpu
from jax.experimental.pallas import tpu_sc as plsc

N_NODES = 10000
N_EDGES = 320000
D = 128
RBF = 16

NC = 2
NS = 16
NW = NC * NS
EPW = N_EDGES // NW
C = 40
K = EPW // C
N_PAD = 10240
ROWS_PER_TILE = N_PAD // NS

EB = 2560


def _linear_kernel(x_ref, w_ref, b_ref, o_ref):
    o_ref[...] = (
        jnp.dot(x_ref[...], w_ref[...], preferred_element_type=jnp.float32)
        + b_ref[...]
    )


def _node_linear(x, w, b):
    return pl.pallas_call(
        _linear_kernel,
        out_shape=jax.ShapeDtypeStruct((x.shape[0], w.shape[1]), jnp.float32),
    )(x, w, b.reshape(1, -1))


def _sum_linear_kernel(p_ref, w_ref, b_ref, o_ref):
    s = p_ref[0, :N_NODES, :] + p_ref[1, :N_NODES, :]
    o_ref[...] = (
        jnp.dot(s, w_ref[...], preferred_element_type=jnp.float32) + b_ref[...]
    )


def _final_linear(partials, w, b):
    return pl.pallas_call(
        _sum_linear_kernel,
        out_shape=jax.ShapeDtypeStruct((N_NODES, w.shape[1]), jnp.float32),
    )(partials, w, b.reshape(1, -1))


def _filter_kernel(rbf_ref, w1_ref, b1_ref, w2_ref, b2_ref, o_ref):
    h = (
        jnp.dot(rbf_ref[...], w1_ref[...], preferred_element_type=jnp.float32)
        + b1_ref[...]
    )
    h = h * jax.nn.sigmoid(h)
    o_ref[...] = (
        jnp.dot(h, w2_ref[...], preferred_element_type=jnp.float32) + b2_ref[...]
    )


def _filter_net(edge_rbf, fn1_w, fn1_b, fn2_w, fn2_b):
    grid = (N_EDGES // EB,)
    return pl.pallas_call(
        _filter_kernel,
        grid=grid,
        in_specs=[
            pl.BlockSpec((EB, RBF), lambda i: (i, 0)),
            pl.BlockSpec((RBF, D), lambda i: (0, 0)),
            pl.BlockSpec((1, D), lambda i: (0, 0)),
            pl.BlockSpec((D, D), lambda i: (0, 0)),
            pl.BlockSpec((1, D), lambda i: (0, 0)),
        ],
        out_specs=pl.BlockSpec((EB, D), lambda i: (i, 0)),
        out_shape=jax.ShapeDtypeStruct((N_EDGES, D), jnp.float32),
    )(edge_rbf, fn1_w, fn1_b.reshape(1, D), fn2_w, fn2_b.reshape(1, D))


NBUF = 4
PF = 2
IRING = 8
UNROLL = IRING


def _sc_body(x1_hbm, idx_hbm, filt_hbm, zeros_hbm, out_hbm, idxv, *rest):
    gbufs = rest[0:NBUF]
    fbufs = rest[NBUF:2 * NBUF]
    acc = rest[2 * NBUF]
    g_sems = rest[2 * NBUF + 1:3 * NBUF + 1]
    f_sems = rest[3 * NBUF + 1:4 * NBUF + 1]
    s_sems = rest[4 * NBUF + 1:5 * NBUF + 1]
    i_sems = rest[5 * NBUF + 1:5 * NBUF + 1 + IRING]

    cid = lax.axis_index("c")
    sid = lax.axis_index("s")
    wid = sid * NC + cid
    base0 = wid * EPW

    pltpu.sync_copy(
        zeros_hbm.at[pl.ds(sid * ROWS_PER_TILE, ROWS_PER_TILE)],
        acc.at[pl.ds(sid * ROWS_PER_TILE, ROWS_PER_TILE)],
    )
    plsc.subcore_barrier()

    def idx_fire(i, q):
        pltpu.async_copy(idx_hbm.at[wid, i], idxv.at[q], i_sems[q])

    def idx_wait(i, q):
        pltpu.make_async_copy(idx_hbm.at[wid, i], idxv.at[q], i_sems[q]).wait()

    def gf_fire(i, b, q):
        pltpu.async_copy(x1_hbm.at[idxv.at[q, 1]], gbufs[b], g_sems[b])
        pltpu.async_copy(
            filt_hbm.at[pl.ds(base0 + i * C, C)], fbufs[b], f_sems[b])

    def s_wait(b):
        pltpu.make_async_copy(gbufs[b], acc.at[idxv.at[0, 0]], s_sems[b]).wait()

    def proc(i, u):
        b = u % NBUF
        b2 = (u + PF) % NBUF
        q2 = (u + PF) % IRING
        q4 = (u + PF + 2) % IRING

        @pl.when(i + PF + 2 < K)
        def _():
            idx_fire(i + PF + 2, q4)

        @pl.when(i + PF < K)
        def _():
            @pl.when(i >= PF)
            def _():
                s_wait(b2)
            idx_wait(i + PF, q2)
            gf_fire(i + PF, b2, q2)

        pltpu.make_async_copy(
            x1_hbm.at[idxv.at[u, 1]], gbufs[b], g_sems[b]).wait()
        pltpu.make_async_copy(
            filt_hbm.at[pl.ds(base0 + i * C, C)], fbufs[b], f_sems[b]).wait()

        gb, fb = gbufs[b], fbufs[b]

        def mul2(r, c2):
            r0 = r * 2
            for rr in range(2):
                for j in range(D // 16):
                    s = pl.ds(j * 16, 16)
                    gb[r0 + rr, s] = gb[r0 + rr, s] * fb[r0 + rr, s]
            return c2

        lax.fori_loop(0, C // 2, mul2, 0)
        pltpu.async_copy(gb, acc.at[idxv.at[u, 0]], s_sems[b], add=True)

    for j in range(PF + 2):
        idx_fire(j, j)
    for j in range(PF):
        idx_wait(j, j)
        gf_fire(j, j % NBUF, j)

    def outer(i2, carry):
        for u in range(UNROLL):
            proc(i2 * UNROLL + u, u)
        return carry

    n_outer = K // UNROLL
    lax.fori_loop(0, n_outer, outer, 0)
    for i in range(n_outer * UNROLL, K):
        proc(jnp.int32(i), i % UNROLL)
    for b in range(NBUF):
        s_wait(b)

    plsc.subcore_barrier()

    pltpu.sync_copy(
        acc.at[pl.ds(sid * ROWS_PER_TILE, ROWS_PER_TILE)],
        out_hbm.at[cid, pl.ds(sid * ROWS_PER_TILE, ROWS_PER_TILE)],
    )


def _sc_gather_mul_scatter(x1, edge_idx, filt, zeros):
    mesh = plsc.VectorSubcoreMesh(core_axis_name="c", subcore_axis_name="s")
    f = functools.partial(
        pl.kernel,
        mesh=mesh,
        out_type=jax.ShapeDtypeStruct((NC, N_PAD, D), jnp.float32),
        scratch_types=[
            pltpu.VMEM((IRING, 2, C), jnp.int32),
            *[pltpu.VMEM((C, D), jnp.float32) for _ in range(2 * NBUF)],
            pltpu.VMEM_SHARED((N_PAD, D), jnp.float32),
            *[pltpu.SemaphoreType.DMA for _ in range(3 * NBUF + IRING)],
        ],
    )(_sc_body)
    idx3 = edge_idx.reshape(2, NW, K, C).transpose(1, 2, 0, 3)
    return f(x1, idx3, filt, zeros)


def kernel(x, edge_index, edge_rbf, lin1_w, lin1_b, lin2_w, lin2_b,
           fn1_w, fn1_b, fn2_w, fn2_b):
    ei = edge_index.astype(jnp.int32)
    x1 = _node_linear(x, lin1_w, lin1_b)
    filt = _filter_net(edge_rbf, fn1_w, fn1_b, fn2_w, fn2_b)
    zeros = jnp.zeros((N_PAD, D), dtype=jnp.float32)
    partials = _sc_gather_mul_scatter(x1, ei, filt, zeros)
    return _final_linear(partials, lin2_w, lin2_b)
